# Initial kernel scaffold; baseline (speedup 1.0000x reference)
import jax
import jax.numpy as jnp
from jax.experimental import pallas as pl


def kernel(atom_types, system_indices, weights, type_to_index):
    raise NotImplementedError("write your pallas kernel here")


# speedup vs baseline: 102.4073x; 102.4073x over previous
import functools

import jax
import jax.numpy as jnp
from jax import lax
from jax.experimental import pallas as pl
from jax.experimental.pallas import tpu as pltpu
from jax.experimental.pallas import tpu_sc as plsc

N_ATOMS = 500000
N_TYPES = 100
N_PROPS = 128
N_SYSTEMS = 2048

NC = 2
NS = 16
LANES = 16
NW = NC * NS

STRIDE = 15584
CHUNK = 16896
NROWS = CHUNK // 128
NVREG = CHUNK // LANES
HBINS = N_SYSTEMS * 128
ZSLICE = HBINS // NS

assert (NW - 1) * STRIDE + CHUNK == N_ATOMS
assert STRIDE % 8 == 0 and CHUNK % 128 == 0


def _sc_hist_body(types_hbm, sys_hbm, out_hbm, types_v, sys_v, idx_v, val_v,
                  buf_v, shared):
    c = lax.axis_index("c")
    s = lax.axis_index("s")
    wid = c * NS + s
    base = wid * STRIDE

    def zero_body(k, _):
        buf_v[pl.ds(k * LANES, LANES)] = jnp.zeros((LANES,), jnp.float32)
        return _
    lax.fori_loop(0, ZSLICE // LANES, zero_body, None)
    pltpu.sync_copy(buf_v, shared.at[pl.ds(s * ZSLICE, ZSLICE)])

    pltpu.sync_copy(types_hbm.at[pl.ds(base, CHUNK)], types_v)
    pltpu.sync_copy(sys_hbm.at[pl.ds(base, CHUNK)], sys_v)

    limit = jnp.where(wid == NW - 1, CHUNK, STRIDE)

    def comp_body(i, _):
        t = types_v[pl.ds(i * LANES, LANES)]
        sy = sys_v[pl.ds(i * LANES, LANES)]
        comb = sy * 128 + t
        pos = i * LANES + lax.iota(jnp.int32, LANES)
        v = jnp.where(pos < limit, 1.0, 0.0).astype(jnp.float32)
        row = i // 8
        col = (i % 8) * LANES
        idx_v[row, pl.ds(col, LANES)] = comb
        val_v[row, pl.ds(col, LANES)] = v
        return _
    lax.fori_loop(0, NVREG, comp_body, None)

    plsc.subcore_barrier()

    def scat_body(j, _):
        pltpu.sync_copy(val_v.at[j], shared.at[idx_v.at[j]], add=True)
        return _
    lax.fori_loop(0, NROWS, scat_body, None)

    plsc.subcore_barrier()

    pltpu.sync_copy(shared.at[pl.ds(s * ZSLICE, ZSLICE)], buf_v)
    pltpu.sync_copy(buf_v, out_hbm.at[c, pl.ds(s * ZSLICE, ZSLICE)])


def _sc_hist(atom_types, system_indices):
    mesh = plsc.VectorSubcoreMesh(core_axis_name="c", subcore_axis_name="s")
    return pl.kernel(
        _sc_hist_body,
        out_type=jax.ShapeDtypeStruct((NC, HBINS), jnp.float32),
        mesh=mesh,
        scratch_types=[
            pltpu.VMEM((CHUNK,), jnp.int32),
            pltpu.VMEM((CHUNK,), jnp.int32),
            pltpu.VMEM((NROWS, 128), jnp.int32),
            pltpu.VMEM((NROWS, 128), jnp.float32),
            pltpu.VMEM((ZSLICE,), jnp.float32),
            pltpu.VMEM_SHARED((HBINS,), jnp.float32),
        ],
    )(atom_types, system_indices)


def _tc_matmul_body(hist_ref, w_ref, t2i_ref, out_ref):
    h = hist_ref[0] + hist_ref[1]
    r = lax.broadcasted_iota(jnp.int32, (128, 128), 1)
    m = (t2i_ref[...] == r).astype(jnp.float32)
    w_eff = jnp.dot(m, w_ref[...], preferred_element_type=jnp.float32)
    out_ref[...] = jnp.dot(h, w_eff, preferred_element_type=jnp.float32)


def _tc_matmul(hist, w_pad, t2i_pad):
    return pl.pallas_call(
        _tc_matmul_body,
        out_shape=jax.ShapeDtypeStruct((N_SYSTEMS, N_PROPS), jnp.float32),
        in_specs=[
            pl.BlockSpec(memory_space=pltpu.VMEM),
            pl.BlockSpec(memory_space=pltpu.VMEM),
            pl.BlockSpec(memory_space=pltpu.VMEM),
        ],
        out_specs=pl.BlockSpec(memory_space=pltpu.VMEM),
    )(hist, w_pad, t2i_pad)


def kernel(atom_types, system_indices, weights, type_to_index):
    hist = _sc_hist(atom_types, system_indices)
    hist = hist.reshape(NC, N_SYSTEMS, 128)
    w_pad = jnp.pad(weights, ((0, 128 - N_TYPES), (0, 0)))
    t2i_pad = jnp.pad(type_to_index, (0, 128 - N_TYPES)).reshape(128, 1)
    return _tc_matmul(hist, w_pad, t2i_pad)
